# baseline (speedup 1.0000x reference)
import functools

import jax
import jax.numpy as jnp
from jax import lax
from jax.experimental import pallas as pl
from jax.experimental.pallas import tpu as pltpu
from jax.experimental.pallas import tpu_sc as plsc

NUM_CORES = 2
NUM_SUBCORES = 16
NUM_WORKERS = NUM_CORES * NUM_SUBCORES
LANES = 16

CHUNK_ROWS = 8


@functools.partial(jax.jit, static_argnames=("batch", "seq", "d"))
def _sc_broadcast_add(x_flat, pe_flat, *, batch, seq, d):
    rows_per_w = seq // NUM_WORKERS
    chunk = CHUNK_ROWS * d
    n_chunks = rows_per_w // CHUNK_ROWS
    seq_elems = seq * d

    mesh = plsc.VectorSubcoreMesh(
        core_axis_name="c", subcore_axis_name="s"
    )

    @functools.partial(
        pl.kernel,
        out_type=jax.ShapeDtypeStruct((batch * seq * d,), jnp.float32),
        mesh=mesh,
        scratch_types=[
            pltpu.VMEM((chunk,), jnp.float32),
            pltpu.VMEM((chunk,), jnp.float32),
        ],
    )
    def body(x_hbm, pe_hbm, out_hbm, pe_buf, x_buf):
        cid = lax.axis_index("c")
        sid = lax.axis_index("s")
        wid = sid * NUM_CORES + cid
        base = wid * rows_per_w * d

        def chunk_step(ci, _):
            off = base + ci * chunk
            pltpu.sync_copy(pe_hbm.at[pl.ds(off, chunk)], pe_buf)

            def batch_step(b, _):
                xoff = b * seq_elems + off
                pltpu.sync_copy(x_hbm.at[pl.ds(xoff, chunk)], x_buf)

                def add_step(i, _):
                    sl = pl.ds(i * LANES, LANES)
                    x_buf[sl] = x_buf[sl] + pe_buf[sl]
                    return 0

                lax.fori_loop(0, chunk // LANES, add_step, 0)
                pltpu.sync_copy(x_buf, out_hbm.at[pl.ds(xoff, chunk)])
                return 0

            lax.fori_loop(0, batch, batch_step, 0)
            return 0

        lax.fori_loop(0, n_chunks, chunk_step, 0)

    return body(x_flat, pe_flat)


def kernel(x, pos_embedding):
    batch, seq, d = x.shape
    pe = pos_embedding[:seq]
    out_flat = _sc_broadcast_add(
        x.reshape(-1), pe.reshape(-1), batch=batch, seq=seq, d=d
    )
    return out_flat.reshape(batch, seq, d)


# speedup vs baseline: 1.7599x; 1.7599x over previous
import functools

import jax
import jax.numpy as jnp
from jax import lax
from jax.experimental import pallas as pl
from jax.experimental.pallas import tpu as pltpu
from jax.experimental.pallas import tpu_sc as plsc

NUM_CORES = 2
NUM_SUBCORES = 16
NUM_WORKERS = NUM_CORES * NUM_SUBCORES
LANES = 16

CHUNK_ROWS = 8
UNROLL = 8


@functools.partial(jax.jit, static_argnames=("batch", "seq", "d"))
def _sc_broadcast_add(x_flat, pe_flat, *, batch, seq, d):
    rows_per_w = seq // NUM_WORKERS
    chunk = CHUNK_ROWS * d
    n_chunks = rows_per_w // CHUNK_ROWS
    n_steps = n_chunks * batch
    seq_elems = seq * d
    vecs_per_iter = UNROLL * LANES
    n_add_iters = chunk // vecs_per_iter

    mesh = plsc.VectorSubcoreMesh(core_axis_name="c", subcore_axis_name="s")

    @functools.partial(
        pl.kernel,
        out_type=jax.ShapeDtypeStruct((batch * seq * d,), jnp.float32),
        mesh=mesh,
        scratch_types=[
            pltpu.VMEM((chunk,), jnp.float32),
            pltpu.VMEM((chunk,), jnp.float32),
            pltpu.VMEM((chunk,), jnp.float32),
            pltpu.VMEM((chunk,), jnp.float32),
            pltpu.VMEM((chunk,), jnp.float32),
            pltpu.VMEM((chunk,), jnp.float32),
            pltpu.SemaphoreType.DMA,
            pltpu.SemaphoreType.DMA,
            pltpu.SemaphoreType.DMA,
            pltpu.SemaphoreType.DMA,
            pltpu.SemaphoreType.DMA,
            pltpu.SemaphoreType.DMA,
        ],
    )
    def body(x_hbm, pe_hbm, out_hbm, pe0, pe1, xb0, xb1, ob0, ob1,
             spe0, spe1, sx0, sx1, so0, so1):
        cid = lax.axis_index("c")
        sid = lax.axis_index("s")
        wid = sid * NUM_CORES + cid
        base = wid * rows_per_w * d

        pe_bufs, x_bufs, o_bufs = (pe0, pe1), (xb0, xb1), (ob0, ob1)
        pe_sems, x_sems, o_sems = (spe0, spe1), (sx0, sx1), (so0, so1)

        pe_descs = [None] * n_chunks
        x_descs = [None] * n_steps
        o_descs = [None] * n_steps

        def issue_loads(s):
            c, b = s // batch, s % batch
            off = base + c * chunk
            if b == 0:
                pe_descs[c] = pltpu.async_copy(
                    pe_hbm.at[pl.ds(off, chunk)], pe_bufs[c % 2],
                    pe_sems[c % 2])
            x_descs[s] = pltpu.async_copy(
                x_hbm.at[pl.ds(b * seq_elems + off, chunk)],
                x_bufs[s % 2], x_sems[s % 2])

        def process(s):
            c, b = s // batch, s % batch
            off = base + c * chunk
            if b == 0:
                pe_descs[c].wait()
            x_descs[s].wait()
            if s >= 2:
                o_descs[s - 2].wait()
            pe_buf, x_buf, o_buf = pe_bufs[c % 2], x_bufs[s % 2], o_bufs[s % 2]

            def add_iter(i, _):
                ib = i * vecs_per_iter
                for j in range(UNROLL):
                    sl = pl.ds(ib + j * LANES, LANES)
                    o_buf[sl] = x_buf[sl] + pe_buf[sl]
                return 0

            lax.fori_loop(0, n_add_iters, add_iter, 0, unroll=False)
            o_descs[s] = pltpu.async_copy(
                o_buf, out_hbm.at[pl.ds(b * seq_elems + off, chunk)],
                o_sems[s % 2])

        issue_loads(0)
        for s in range(1, n_steps):
            issue_loads(s)
            process(s - 1)
        process(n_steps - 1)
        o_descs[n_steps - 2].wait()
        o_descs[n_steps - 1].wait()

    return body(x_flat, pe_flat)


def kernel(x, pos_embedding):
    batch, seq, d = x.shape
    pe = pos_embedding[:seq]
    out_flat = _sc_broadcast_add(
        x.reshape(-1), pe.reshape(-1), batch=batch, seq=seq, d=d
    )
    return out_flat.reshape(batch, seq, d)
